# baseline (speedup 1.0000x reference)
import functools

import jax
import jax.numpy as jnp
from jax import lax
from jax.experimental import pallas as pl
from jax.experimental.pallas import tpu as pltpu
from jax.experimental.pallas import tpu_sc as plsc

_NC = 2
_NS = 16
_NW = _NC * _NS
_PAD = 128


@functools.partial(jax.jit, static_argnames=("batch", "hist", "dim"))
def _embed(xs, tpad, batch, hist, dim):
    rows_per_w = batch // _NW
    n_pairs = rows_per_w // 2
    mesh = plsc.VectorSubcoreMesh(core_axis_name="c", subcore_axis_name="s")

    @functools.partial(
        pl.kernel,
        mesh=mesh,
        out_type=jax.ShapeDtypeStruct((batch, hist, dim), jnp.float32),
        scratch_types=[
            pltpu.VMEM((hist,), jnp.int32),
            pltpu.VMEM((hist,), jnp.int32),
            pltpu.VMEM((hist, _PAD), jnp.float32),
            pltpu.VMEM((hist, _PAD), jnp.float32),
            pltpu.VMEM((hist, dim), jnp.float32),
            pltpu.VMEM((hist, dim), jnp.float32),
            pltpu.SemaphoreType.DMA,
            pltpu.SemaphoreType.DMA,
            pltpu.SemaphoreType.DMA,
            pltpu.SemaphoreType.DMA,
            pltpu.SemaphoreType.DMA,
            pltpu.SemaphoreType.DMA,
        ],
    )
    def k(xs_hbm, tpad_hbm, out_hbm, i0, i1, raw0, raw1, cmp0, cmp1,
          is0, is1, gs0, gs1, os0, os1):
        wid = lax.axis_index("s") * _NC + lax.axis_index("c")
        base = wid * rows_per_w

        def i_copy(j, ibuf, sem):
            return pltpu.make_async_copy(xs_hbm.at[base + j], ibuf, sem)

        def g_copy(ibuf, raw, sem):
            return pltpu.make_async_copy(tpad_hbm.at[ibuf], raw, sem)

        def o_copy(j, cmp, sem):
            return pltpu.make_async_copy(cmp, out_hbm.at[base + j], sem)

        def strip(raw, cmp):
            def tstep(t8, carry):
                t0 = t8 * 8
                for dt in range(8):
                    for c in range(dim // 16):
                        cmp[t0 + dt, pl.ds(c * 16, 16)] = (
                            raw[t0 + dt, pl.ds(c * 16, 16)])
                return carry
            lax.fori_loop(0, hist // 8, tstep, 0)

        i_copy(0, i0, is0).start()
        i_copy(1, i1, is1).start()

        def body(p, carry):
            je = 2 * p
            jo = je + 1

            i_copy(je, i0, is0).wait()
            g_copy(i0, raw0, gs0).start()
            i_copy(jo, i1, is1).wait()
            g_copy(i1, raw1, gs1).start()

            g_copy(i0, raw0, gs0).wait()

            @pl.when(p < n_pairs - 1)
            def _():
                i_copy(je + 2, i0, is0).start()

            @pl.when(p > 0)
            def _():
                o_copy(je - 2, cmp0, os0).wait()
            strip(raw0, cmp0)
            o_copy(je, cmp0, os0).start()

            g_copy(i1, raw1, gs1).wait()

            @pl.when(p < n_pairs - 1)
            def _():
                i_copy(jo + 2, i1, is1).start()

            @pl.when(p > 0)
            def _():
                o_copy(jo - 2, cmp1, os1).wait()
            strip(raw1, cmp1)
            o_copy(jo, cmp1, os1).start()
            return carry

        lax.fori_loop(0, n_pairs, body, 0)
        o_copy(rows_per_w - 2, cmp0, os0).wait()
        o_copy(rows_per_w - 1, cmp1, os1).wait()

    return k(xs, tpad)


def kernel(xs, table):
    b, t = xs.shape
    v, dim = table.shape
    tpad = jnp.pad(table, ((0, 0), (0, _PAD - dim)))
    return _embed(xs.astype(jnp.int32), tpad, batch=b, hist=t, dim=dim)
